# baseline (speedup 1.0000x reference)
import functools

import jax
import jax.numpy as jnp
from jax import lax
from jax.experimental import pallas as pl
from jax.experimental.pallas import tpu as pltpu
from jax.experimental.pallas import tpu_sc as plsc

NUM_CLASSES = 1000000
EMBED_DIM = 16
BATCH = 16384

_INFO = plsc.get_sparse_core_info()
_NC, _NS = _INFO.num_cores, _INFO.num_subcores
_NW = _NC * _NS
_B_PER_W = BATCH // _NW
_CHUNK = 128
_NCHUNK = _B_PER_W // _CHUNK

_mesh = plsc.VectorSubcoreMesh(core_axis_name="c", subcore_axis_name="s")


@functools.partial(
    pl.kernel,
    mesh=_mesh,
    compiler_params=pltpu.CompilerParams(use_tc_tiling_on_sc=False),
    out_type=jax.ShapeDtypeStruct((BATCH, EMBED_DIM), jnp.float32),
    scratch_types=[
        pltpu.VMEM((_NCHUNK, _CHUNK), jnp.int32),
        pltpu.VMEM((_B_PER_W, EMBED_DIM), jnp.float32),
        pltpu.SemaphoreType.DMA,
    ],
)
def _gather_kernel(labels_hbm, table_hbm, out_hbm, idx_v, rows_v, sem):
    wid = lax.axis_index("s") * _NC + lax.axis_index("c")
    base = wid * _NCHUNK
    pltpu.sync_copy(labels_hbm.at[pl.ds(base, _NCHUNK)], idx_v)
    copies = []
    for j in range(_NCHUNK):
        copies.append(
            pltpu.async_copy(
                table_hbm.at[idx_v.at[j]],
                rows_v.at[pl.ds(j * _CHUNK, _CHUNK)],
                sem,
            )
        )
    for c in copies:
        c.wait()
    pltpu.sync_copy(rows_v, out_hbm.at[pl.ds(base * _CHUNK, _B_PER_W)])


def kernel(labels, table):
    labels2d = labels.astype(jnp.int32).reshape(BATCH // _CHUNK, _CHUNK)
    return _gather_kernel(labels2d, table)


# speedup vs baseline: 5.9358x; 5.9358x over previous
import functools

import jax
import jax.numpy as jnp
from jax import lax
from jax.experimental import pallas as pl
from jax.experimental.pallas import tpu as pltpu
from jax.experimental.pallas import tpu_sc as plsc

NUM_CLASSES = 1000000
EMBED_DIM = 16
BATCH = 16384

_INFO = plsc.get_sparse_core_info()
_NC, _NS = _INFO.num_cores, _INFO.num_subcores
_NW = _NC * _NS
_B_PER_W = BATCH // _NW
_GRP = 16
_NGRP = _B_PER_W // _GRP

_mesh = plsc.VectorSubcoreMesh(core_axis_name="c", subcore_axis_name="s")


@functools.partial(
    pl.kernel,
    mesh=_mesh,
    compiler_params=pltpu.CompilerParams(needs_layout_passes=False),
    out_type=jax.ShapeDtypeStruct((EMBED_DIM, BATCH), jnp.float32),
    scratch_types=[
        pltpu.VMEM((8, 128), jnp.int32),
        pltpu.VMEM((2, _GRP, EMBED_DIM, 128), jnp.float32),
        pltpu.VMEM((EMBED_DIM, _B_PER_W), jnp.float32),
        pltpu.SemaphoreType.DMA,
        pltpu.SemaphoreType.DMA,
    ],
)
def _gather_kernel(labels_hbm, tablet_hbm, outt_hbm, lbl_v, slots, out_v,
                   sem_a, sem_b):
    wid = lax.axis_index("s") * _NC + lax.axis_index("c")
    base = wid * _B_PER_W
    pltpu.sync_copy(labels_hbm.at[pl.ds((wid // 2) * 8, 8), pl.ds(0, 128)], lbl_v)
    row0 = (wid % 2) * 4
    lanes = lax.iota(jnp.int32, 16)
    sems = (sem_a, sem_b)

    def _group_labels(g):
        return lbl_v[row0 + g // 8, pl.ds((g % 8) * 16, 16)]

    def _enqueue_group(g, parity, sem):
        lbl16 = _group_labels(g)
        cb16 = (lbl16 >> 7) << 7
        for b in range(_GRP):
            cbase = pl.multiple_of(cb16[b], 128)
            pltpu.async_copy(
                tablet_hbm.at[pl.ds(0, EMBED_DIM), pl.ds(cbase, 128)],
                slots.at[parity, b],
                sem,
            )

    def _extract_group(g, parity):
        lbl16 = _group_labels(g)
        coff16 = lbl16 & 127
        for b in range(_GRP):
            coff = jnp.full((16,), coff16[b], jnp.int32)
            vals = plsc.load_gather(slots.at[parity, b], [lanes, coff])
            plsc.store_scatter(
                out_v, [lanes, jnp.full((16,), g * _GRP + b, jnp.int32)], vals
            )

    def _wait_group(parity, sem):
        for b in range(_GRP):
            pltpu.make_async_copy(
                tablet_hbm.at[pl.ds(0, EMBED_DIM), pl.ds(0, 128)],
                slots.at[parity, b],
                sem,
            ).wait()

    _enqueue_group(0, 0, sem_a)
    _enqueue_group(1, 1, sem_b)

    @pl.loop(0, _NGRP // 2)
    def _body(g2):
        for parity in range(2):
            g = g2 * 2 + parity
            sem = sems[parity]
            _wait_group(parity, sem)
            _extract_group(g, parity)

            @pl.when(g + 2 < _NGRP)
            def _():
                _enqueue_group(g + 2, parity, sem)

    pltpu.sync_copy(
        out_v, outt_hbm.at[pl.ds(0, EMBED_DIM), pl.ds(base, _B_PER_W)]
    )


def kernel(labels, table):
    labels2d = labels.astype(jnp.int32).reshape(BATCH // 128, 128)
    outt = _gather_kernel(labels2d, table.T)
    return outt.T


# speedup vs baseline: 6.3460x; 1.0691x over previous
import functools

import jax
import jax.numpy as jnp
from jax import lax
from jax.experimental import pallas as pl
from jax.experimental.pallas import tpu as pltpu
from jax.experimental.pallas import tpu_sc as plsc

NUM_CLASSES = 1000000
EMBED_DIM = 16
BATCH = 16384

_INFO = plsc.get_sparse_core_info()
_NC, _NS = _INFO.num_cores, _INFO.num_subcores
_NW = _NC * _NS
_B_PER_W = BATCH // _NW
_G = 16
_P = 3
_NGRP = _B_PER_W // _G

_mesh = plsc.VectorSubcoreMesh(core_axis_name="c", subcore_axis_name="s")


@functools.partial(
    pl.kernel,
    mesh=_mesh,
    compiler_params=pltpu.CompilerParams(needs_layout_passes=False),
    out_type=jax.ShapeDtypeStruct((EMBED_DIM, BATCH), jnp.float32),
    scratch_types=[
        pltpu.VMEM((8, 128), jnp.int32),
        pltpu.VMEM((_P, _G, EMBED_DIM, 128), jnp.float32),
        pltpu.VMEM((EMBED_DIM, _B_PER_W), jnp.float32),
    ]
    + [pltpu.SemaphoreType.DMA] * _P,
)
def _gather_kernel(labels_hbm, tablet_hbm, outt_hbm, lbl_v, slots, out_v,
                   *sems):
    wid = lax.axis_index("s") * _NC + lax.axis_index("c")
    base = wid * _B_PER_W
    pltpu.sync_copy(labels_hbm.at[pl.ds((wid // 2) * 8, 8), pl.ds(0, 128)], lbl_v)
    row0 = (wid % 2) * 4
    lanes = lax.iota(jnp.int32, 16)
    per_row = 128 // _G

    def _group_labels(g):
        return lbl_v[row0 + g // per_row, pl.ds((g % per_row) * _G, _G)]

    def _enqueue_group(g, p, sem):
        lblg = _group_labels(g)
        cbg = (lblg >> 7) << 7
        for b in range(_G):
            cbase = pl.multiple_of(cbg[b], 128)
            pltpu.async_copy(
                tablet_hbm.at[pl.ds(0, EMBED_DIM), pl.ds(cbase, 128)],
                slots.at[p, b],
                sem,
            )

    def _extract_group(g, p):
        lblg = _group_labels(g)
        coffg = lblg & 127
        for b in range(_G):
            coff = jnp.full((16,), coffg[b], jnp.int32)
            vals = plsc.load_gather(slots.at[p, b], [lanes, coff])
            plsc.store_scatter(
                out_v, [lanes, jnp.full((16,), g * _G + b, jnp.int32)], vals
            )

    def _wait_group(p, sem):
        for b in range(_G):
            pltpu.make_async_copy(
                tablet_hbm.at[pl.ds(0, EMBED_DIM), pl.ds(0, 128)],
                slots.at[p, b],
                sem,
            ).wait()

    for p in range(_P):
        _enqueue_group(p, p, sems[p])

    @pl.loop(0, _NGRP // _P)
    def _body(gp):
        for p in range(_P):
            g = gp * _P + p
            _wait_group(p, sems[p])
            _extract_group(g, p)

            @pl.when(g + _P < _NGRP)
            def _():
                _enqueue_group(g + _P, p, sems[p])

    for g in range((_NGRP // _P) * _P, _NGRP):
        p = g % _P
        _wait_group(p, sems[p])
        _extract_group(g, p)

    pltpu.sync_copy(
        out_v, outt_hbm.at[pl.ds(0, EMBED_DIM), pl.ds(base, _B_PER_W)]
    )


def kernel(labels, table):
    labels2d = labels.astype(jnp.int32).reshape(BATCH // 128, 128)
    outt = _gather_kernel(labels2d, table.T)
    return outt.T


# speedup vs baseline: 6.4047x; 1.0093x over previous
import functools

import jax
import jax.numpy as jnp
from jax import lax
from jax.experimental import pallas as pl
from jax.experimental.pallas import tpu as pltpu
from jax.experimental.pallas import tpu_sc as plsc

NUM_CLASSES = 1000000
EMBED_DIM = 16
BATCH = 16384

_INFO = plsc.get_sparse_core_info()
_NC, _NS = _INFO.num_cores, _INFO.num_subcores
_NW = _NC * _NS
_B_PER_W = BATCH // _NW
_G = 16
_P = 3
_NGRP = _B_PER_W // _G

_mesh = plsc.VectorSubcoreMesh(core_axis_name="c", subcore_axis_name="s")


@functools.partial(
    pl.kernel,
    mesh=_mesh,
    compiler_params=pltpu.CompilerParams(
        needs_layout_passes=False,
        disable_bounds_checks=True,
        disable_semaphore_checks=True,
    ),
    out_type=jax.ShapeDtypeStruct((EMBED_DIM, BATCH), jnp.float32),
    scratch_types=[
        pltpu.VMEM((8, 128), jnp.int32),
        pltpu.VMEM((_P, _G, EMBED_DIM, 128), jnp.float32),
        pltpu.VMEM((EMBED_DIM, _B_PER_W), jnp.float32),
    ]
    + [pltpu.SemaphoreType.DMA] * _P,
)
def _gather_kernel(labels_hbm, tablet_hbm, outt_hbm, lbl_v, slots, out_v,
                   *sems):
    wid = lax.axis_index("s") * _NC + lax.axis_index("c")
    base = wid * _B_PER_W
    pltpu.sync_copy(labels_hbm.at[pl.ds((wid // 2) * 8, 8), pl.ds(0, 128)], lbl_v)
    row0 = (wid % 2) * 4
    lanes = lax.iota(jnp.int32, 16)
    per_row = 128 // _G

    def _group_labels(g):
        return lbl_v[row0 + g // per_row, pl.ds((g % per_row) * _G, _G)]

    def _enqueue_group(g, p, sem):
        lblg = _group_labels(g)
        cbg = (lblg >> 7) << 7
        for b in range(_G):
            cbase = pl.multiple_of(cbg[b], 128)
            pltpu.async_copy(
                tablet_hbm.at[pl.ds(0, EMBED_DIM), pl.ds(cbase, 128)],
                slots.at[p, b],
                sem,
            )

    def _extract_group(g, p):
        lblg = _group_labels(g)
        coffg = lblg & 127
        for b in range(_G):
            coff = jnp.full((16,), coffg[b], jnp.int32)
            vals = plsc.load_gather(slots.at[p, b], [lanes, coff])
            plsc.store_scatter(
                out_v, [lanes, jnp.full((16,), g * _G + b, jnp.int32)], vals
            )

    def _wait_group(p, sem):
        for b in range(_G):
            pltpu.make_async_copy(
                tablet_hbm.at[pl.ds(0, EMBED_DIM), pl.ds(0, 128)],
                slots.at[p, b],
                sem,
            ).wait()

    for p in range(_P):
        _enqueue_group(p, p, sems[p])

    @pl.loop(0, _NGRP // _P)
    def _body(gp):
        for p in range(_P):
            g = gp * _P + p
            _wait_group(p, sems[p])
            _extract_group(g, p)

            @pl.when(g + _P < _NGRP)
            def _():
                _enqueue_group(g + _P, p, sems[p])

    for g in range((_NGRP // _P) * _P, _NGRP):
        p = g % _P
        _wait_group(p, sems[p])
        _extract_group(g, p)

    pltpu.sync_copy(
        out_v, outt_hbm.at[pl.ds(0, EMBED_DIM), pl.ds(base, _B_PER_W)]
    )


def kernel(labels, table):
    labels2d = labels.astype(jnp.int32).reshape(BATCH // 128, 128)
    outt = _gather_kernel(labels2d, table.T)
    return outt.T


# speedup vs baseline: 6.4760x; 1.0111x over previous
import functools

import jax
import jax.numpy as jnp
from jax import lax
from jax.experimental import pallas as pl
from jax.experimental.pallas import tpu as pltpu
from jax.experimental.pallas import tpu_sc as plsc

NUM_CLASSES = 1000000
EMBED_DIM = 16
BATCH = 16384

_INFO = plsc.get_sparse_core_info()
_NC, _NS = _INFO.num_cores, _INFO.num_subcores
_NW = _NC * _NS
_B_PER_W = BATCH // _NW
_G = 16
_P = 3
_NGRP = _B_PER_W // _G

_mesh = plsc.VectorSubcoreMesh(core_axis_name="c", subcore_axis_name="s")


@functools.partial(
    pl.kernel,
    mesh=_mesh,
    compiler_params=pltpu.CompilerParams(
        needs_layout_passes=False,
        disable_bounds_checks=True,
        disable_semaphore_checks=True,
    ),
    out_type=jax.ShapeDtypeStruct((EMBED_DIM, BATCH), jnp.float32),
    scratch_types=[
        pltpu.VMEM((8, 128), jnp.int32),
        pltpu.VMEM((_P, _G, EMBED_DIM, 128), jnp.float32),
        pltpu.VMEM((EMBED_DIM, _B_PER_W), jnp.float32),
    ]
    + [pltpu.SemaphoreType.DMA] * _P,
)
def _gather_kernel(labels_hbm, tablet_hbm, outt_hbm, lbl_v, slots, out_v,
                   *sems):
    wid = lax.axis_index("s") * _NC + lax.axis_index("c")
    base = wid * _B_PER_W
    pltpu.sync_copy(labels_hbm.at[pl.ds((wid // 2) * 8, 8), pl.ds(0, 128)], lbl_v)
    row0 = (wid % 2) * 4
    lanes = lax.iota(jnp.int32, 16)
    per_row = 128 // _G

    def _group_labels(g):
        return lbl_v[row0 + g // per_row, pl.ds((g % per_row) * _G, _G)]

    def _enqueue_group(g, p, sem):
        lblg = _group_labels(g)
        cbg = (lblg >> 7) << 7
        for b in range(_G):
            cbase = pl.multiple_of(cbg[b], 128)
            pltpu.async_copy(
                tablet_hbm.at[pl.ds(0, EMBED_DIM), pl.ds(cbase, 128)],
                slots.at[p, b],
                sem,
            )

    def _wait_extract_group(g, p, sem):
        lblg = _group_labels(g)
        coffg = lblg & 127
        for b in range(_G):
            pltpu.make_async_copy(
                tablet_hbm.at[pl.ds(0, EMBED_DIM), pl.ds(0, 128)],
                slots.at[p, b],
                sem,
            ).wait()
            coff = jnp.full((16,), coffg[b], jnp.int32)
            vals = plsc.load_gather(slots.at[p, b], [lanes, coff])
            plsc.store_scatter(
                out_v, [lanes, jnp.full((16,), g * _G + b, jnp.int32)], vals
            )

    for p in range(_P):
        _enqueue_group(p, p, sems[p])

    @pl.loop(0, _NGRP // _P)
    def _body(gp):
        for p in range(_P):
            g = gp * _P + p
            _wait_extract_group(g, p, sems[p])

            @pl.when(g + _P < _NGRP)
            def _():
                _enqueue_group(g + _P, p, sems[p])

    for g in range((_NGRP // _P) * _P, _NGRP):
        p = g % _P
        _wait_extract_group(g, p, sems[p])

    pltpu.sync_copy(
        out_v, outt_hbm.at[pl.ds(0, EMBED_DIM), pl.ds(base, _B_PER_W)]
    )


def kernel(labels, table):
    labels2d = labels.astype(jnp.int32).reshape(BATCH // 128, 128)
    outt = _gather_kernel(labels2d, table.T)
    return outt.T


# speedup vs baseline: 6.5234x; 1.0073x over previous
import functools

import jax
import jax.numpy as jnp
from jax import lax
from jax.experimental import pallas as pl
from jax.experimental.pallas import tpu as pltpu
from jax.experimental.pallas import tpu_sc as plsc

NUM_CLASSES = 1000000
EMBED_DIM = 16
BATCH = 16384

_INFO = plsc.get_sparse_core_info()
_NC, _NS = _INFO.num_cores, _INFO.num_subcores
_NW = _NC * _NS
_B_PER_W = BATCH // _NW
_G = 16
_P = 3
_NGRP = _B_PER_W // _G

_mesh = plsc.VectorSubcoreMesh(core_axis_name="c", subcore_axis_name="s")


@functools.partial(
    pl.kernel,
    mesh=_mesh,
    compiler_params=pltpu.CompilerParams(
        needs_layout_passes=False,
        disable_bounds_checks=True,
        disable_semaphore_checks=True,
        skip_device_barrier=True,
    ),
    out_type=jax.ShapeDtypeStruct((EMBED_DIM, BATCH), jnp.float32),
    scratch_types=[
        pltpu.VMEM((8, 128), jnp.int32),
        pltpu.VMEM((_P, _G, EMBED_DIM, 128), jnp.float32),
        pltpu.VMEM((EMBED_DIM, _B_PER_W), jnp.float32),
    ]
    + [pltpu.SemaphoreType.DMA] * _P,
)
def _gather_kernel(labels_hbm, tablet_hbm, outt_hbm, lbl_v, slots, out_v,
                   *sems):
    wid = lax.axis_index("s") * _NC + lax.axis_index("c")
    base = wid * _B_PER_W
    pltpu.sync_copy(labels_hbm.at[pl.ds((wid // 2) * 8, 8), pl.ds(0, 128)], lbl_v)
    row0 = (wid % 2) * 4
    lanes = lax.iota(jnp.int32, 16)
    per_row = 128 // _G

    def _group_labels(g):
        return lbl_v[row0 + g // per_row, pl.ds((g % per_row) * _G, _G)]

    def _enqueue_group(g, p, sem):
        lblg = _group_labels(g)
        cbg = (lblg >> 7) << 7
        for b in range(_G):
            cbase = pl.multiple_of(cbg[b], 128)
            pltpu.async_copy(
                tablet_hbm.at[pl.ds(0, EMBED_DIM), pl.ds(cbase, 128)],
                slots.at[p, b],
                sem,
            )

    def _wait_extract_group(g, p, sem):
        lblg = _group_labels(g)
        coffg = lblg & 127
        for b in range(_G):
            pltpu.make_async_copy(
                tablet_hbm.at[pl.ds(0, EMBED_DIM), pl.ds(0, 128)],
                slots.at[p, b],
                sem,
            ).wait()
            coff = jnp.full((16,), coffg[b], jnp.int32)
            vals = plsc.load_gather(slots.at[p, b], [lanes, coff])
            plsc.store_scatter(
                out_v, [lanes, jnp.full((16,), g * _G + b, jnp.int32)], vals
            )

    for p in range(_P):
        _enqueue_group(p, p, sems[p])

    @pl.loop(0, _NGRP // _P)
    def _body(gp):
        for p in range(_P):
            g = gp * _P + p
            _wait_extract_group(g, p, sems[p])

            @pl.when(g + _P < _NGRP)
            def _():
                _enqueue_group(g + _P, p, sems[p])

    for g in range((_NGRP // _P) * _P, _NGRP):
        p = g % _P
        _wait_extract_group(g, p, sems[p])

    pltpu.sync_copy(
        out_v, outt_hbm.at[pl.ds(0, EMBED_DIM), pl.ds(base, _B_PER_W)]
    )


def kernel(labels, table):
    labels2d = labels.astype(jnp.int32).reshape(BATCH // 128, 128)
    outt = _gather_kernel(labels2d, table.T)
    return outt.T
